# baseline (speedup 1.0000x reference)
import functools
import math

import jax
import jax.numpy as jnp
from jax import lax
from jax.experimental import pallas as pl
from jax.experimental.pallas import tpu as pltpu
from jax.experimental.pallas import tpu_sc as plsc

VOCAB = 1000000
D = 64
W = 128
SCALE = math.sqrt(D)
L = 16
NC = 2
NS = 16
NW = NC * NS
C = 128
NBUF = 3

_BLK = 128
_NFULL = VOCAB // _BLK
_TAIL = VOCAB - _NFULL * _BLK
_PER_W = _NFULL // NW
_EXTRA = _NFULL - _PER_W * NW

_fmt_mesh = plsc.VectorSubcoreMesh(core_axis_name="c", subcore_axis_name="s")


@functools.partial(
    pl.kernel,
    out_type=jax.ShapeDtypeStruct((VOCAB // 2, W), jnp.float32),
    mesh=_fmt_mesh,
    compiler_params=pltpu.CompilerParams(
        use_tc_tiling_on_sc=True, needs_layout_passes=False),
    scratch_types=(
        [pltpu.VMEM((D, _BLK), jnp.float32) for _ in range(2)]
        + [pltpu.VMEM((D, _BLK), jnp.float32) for _ in range(2)]
        + [pltpu.SemaphoreType.DMA for _ in range(4)]
    ),
)
def _format_table(lutT_hbm, tail_hbm, tbl_hbm, in0, in1, out0, out1,
                  g0, g1, s0, s1):
    ins, outs, gsems, ssems = (in0, in1), (out0, out1), (g0, g1), (s0, s1)
    wid = lax.axis_index("s") * NC + lax.axis_index("c")
    nblk = jnp.where(wid < _EXTRA, _PER_W + 1, _PER_W)
    start = wid * _PER_W + jnp.minimum(wid, _EXTRA)

    row_idx = [lax.iota(jnp.int32, L) + c * L for c in range(D // L)]

    def in_copy(blk, b):
        return pltpu.make_async_copy(
            lutT_hbm.at[:, pl.ds(blk * _BLK, _BLK)], ins[b], gsems[b])

    def out_copy(blk, b):
        return pltpu.make_async_copy(
            outs[b], tbl_hbm.at[pl.ds(blk * (_BLK // 2), _BLK // 2)],
            ssems[b])

    def transpose_block(b):
        for p in range(_BLK // 2):
            for u in range(2):
                col = jnp.full((L,), 2 * p + u, jnp.int32)
                for c in range(D // L):
                    v = plsc.load_gather(ins[b], [row_idx[c], col])
                    outs[b][p, pl.ds(u * D + c * L, L)] = v * SCALE

    def _pipeline_turn(i, blk, b):
        in_copy(blk, b).wait()

        @pl.when(i + 1 < nblk)
        def _():
            in_copy(blk + 1, 1 - b).start()

        @pl.when(i >= 2)
        def _():
            out_copy(blk - 2, b).wait()

        transpose_block(b)
        out_copy(blk, b).start()

    in_copy(start, 0).start()

    def step(i, _):
        blk = start + i

        @pl.when(lax.rem(i, 2) == 0)
        def _():
            _pipeline_turn(i, blk, 0)

        @pl.when(lax.rem(i, 2) == 1)
        def _():
            _pipeline_turn(i, blk, 1)
        return 0

    lax.fori_loop(0, nblk, step, 0)

    @pl.when(lax.rem(nblk, 2) == 0)
    def _():
        out_copy(start + nblk - 2, 0).wait()
        out_copy(start + nblk - 1, 1).wait()

    @pl.when(lax.rem(nblk, 2) == 1)
    def _():
        out_copy(start + nblk - 2, 1).wait()
        out_copy(start + nblk - 1, 0).wait()

    @pl.when(wid == 0)
    def _():
        pltpu.sync_copy(tail_hbm, out0.at[pl.ds(0, _TAIL // 2)])
        pltpu.sync_copy(out0.at[pl.ds(0, _TAIL // 2)],
                        tbl_hbm.at[pl.ds(_NFULL * (_BLK // 2), _TAIL // 2)])


def _make_gather_kernel(B: int):
    assert B % (NW * C) == 0
    b_per_w = B // NW
    n_chunks = b_per_w // C
    n_main = (n_chunks // NBUF) - 1
    n_peel = n_chunks - NBUF * n_main
    mesh = plsc.VectorSubcoreMesh(core_axis_name="c", subcore_axis_name="s")

    @functools.partial(
        pl.kernel,
        out_type=jax.ShapeDtypeStruct((B, W), jnp.float32),
        mesh=mesh,
        compiler_params=pltpu.CompilerParams(use_tc_tiling_on_sc=False),
        scratch_types=(
            [pltpu.VMEM((b_per_w,), jnp.int32)]
            + [pltpu.VMEM((C, D), jnp.float32) for _ in range(NBUF)]
            + [pltpu.VMEM((C, W), jnp.float32) for _ in range(NBUF)]
            + [pltpu.SemaphoreType.DMA for _ in range(2 * NBUF)]
        ),
    )
    def emb_kernel(x_hbm, lut_hbm, out_hbm, idx_v, *scratch):
        rows_in = scratch[0:NBUF]
        rows_out = scratch[NBUF:2 * NBUF]
        gsem = scratch[2 * NBUF:3 * NBUF]
        ssem = scratch[3 * NBUF:4 * NBUF]

        wid = lax.axis_index("s") * NC + lax.axis_index("c")
        base = wid * b_per_w

        pltpu.sync_copy(x_hbm.at[pl.ds(base, b_per_w)], idx_v)

        def gather_copy(t, b):
            return pltpu.make_async_copy(
                lut_hbm.at[idx_v.at[pl.ds(t * C, C)]], rows_in[b], gsem[b])

        def store_copy(t, b):
            return pltpu.make_async_copy(
                rows_out[b], out_hbm.at[pl.ds(base + t * C, C)], ssem[b])

        def copy_chunk(b):
            def rows(i, _):
                for u in range(4):
                    r = i * 4 + u
                    for c in range(D // L):
                        sl = pl.ds(c * L, L)
                        rows_out[b][r, sl] = rows_in[b][r, sl]
                return 0

            lax.fori_loop(0, C // 4, rows, 0)

        for b in range(NBUF):
            gather_copy(b, b).start()

        def step(s, _):
            for b in range(NBUF):
                t = s * NBUF + b
                gather_copy(t, b).wait()

                @pl.when(s > 0)
                def _():
                    store_copy(t - NBUF, b).wait()

                copy_chunk(b)
                store_copy(t, b).start()
                gather_copy(t + NBUF, b).start()
            return 0

        lax.fori_loop(0, n_main, step, 0)

        for p in range(n_peel):
            t = NBUF * n_main + p
            b = t % NBUF
            gather_copy(t, b).wait()
            if t - NBUF >= 0:
                store_copy(t - NBUF, b).wait()
            copy_chunk(b)
            store_copy(t, b).start()
            if t + NBUF < n_chunks:
                gather_copy(t + NBUF, b).start()

        for t in range(n_chunks - NBUF, n_chunks):
            store_copy(t, t % NBUF).wait()

    return emb_kernel


def kernel(x, lut):
    NB, T = x.shape
    B = NB * T
    tail = (lut[VOCAB - _TAIL:] * SCALE).reshape(_TAIL // 2, W)
    tbl = _format_table(lut.T, tail)
    lut64 = tbl.reshape(VOCAB, D)
    out = _make_gather_kernel(B)(x.reshape(B).astype(jnp.int32), lut64)
    return out[:, :D].reshape(NB, T, D)


# speedup vs baseline: 2.1994x; 2.1994x over previous
import functools
import math

import jax
import jax.numpy as jnp
from jax import lax
from jax.experimental import pallas as pl
from jax.experimental.pallas import tpu as pltpu
from jax.experimental.pallas import tpu_sc as plsc

D = 64
W = 128
SCALE = math.sqrt(D)
L = 16
NC = 2
NS = 16
NW = NC * NS
C = 128
NBUF = 3


def _make_emb_kernel(B: int):
    assert B % (NW * C) == 0
    b_per_w = B // NW
    n_chunks = b_per_w // C
    n_main = (n_chunks // NBUF) - 1
    n_peel = n_chunks - NBUF * n_main
    mesh = plsc.VectorSubcoreMesh(core_axis_name="c", subcore_axis_name="s")

    @functools.partial(
        pl.kernel,
        out_type=jax.ShapeDtypeStruct((B, W), jnp.float32),
        mesh=mesh,
        compiler_params=pltpu.CompilerParams(use_tc_tiling_on_sc=False),
        scratch_types=(
            [pltpu.VMEM((b_per_w,), jnp.int32)]
            + [pltpu.VMEM((C, W), jnp.float32) for _ in range(2 * NBUF)]
            + [pltpu.SemaphoreType.DMA for _ in range(2 * NBUF)]
        ),
    )
    def emb_kernel(x_hbm, lut_hbm, out_hbm, idx_v, *scratch):
        rows_in = scratch[0:NBUF]
        rows_out = scratch[NBUF:2 * NBUF]
        gsem = scratch[2 * NBUF:3 * NBUF]
        ssem = scratch[3 * NBUF:4 * NBUF]

        wid = lax.axis_index("s") * NC + lax.axis_index("c")
        base = wid * b_per_w

        pltpu.sync_copy(x_hbm.at[pl.ds(base, b_per_w)], idx_v)

        def gather_copy(t, b):
            return pltpu.make_async_copy(
                lut_hbm.at[idx_v.at[pl.ds(t * C, C)]], rows_in[b], gsem[b])

        def store_copy(t, b):
            return pltpu.make_async_copy(
                rows_out[b], out_hbm.at[pl.ds(base + t * C, C)], ssem[b])

        def scale_chunk(b):
            def scale_rows(i, _):
                for u in range(4):
                    r = i * 4 + u
                    for c in range(D // L):
                        sl = pl.ds(c * L, L)
                        rows_out[b][r, sl] = rows_in[b][r, sl] * SCALE
                return 0

            lax.fori_loop(0, C // 4, scale_rows, 0)

        for b in range(NBUF):
            gather_copy(b, b).start()

        def step(s, _):
            for b in range(NBUF):
                t = s * NBUF + b
                gather_copy(t, b).wait()

                @pl.when(s > 0)
                def _():
                    store_copy(t - NBUF, b).wait()

                scale_chunk(b)
                store_copy(t, b).start()
                gather_copy(t + NBUF, b).start()
            return 0

        lax.fori_loop(0, n_main, step, 0)

        for p in range(n_peel):
            t = NBUF * n_main + p
            b = t % NBUF
            gather_copy(t, b).wait()
            if t - NBUF >= 0:
                store_copy(t - NBUF, b).wait()
            scale_chunk(b)
            store_copy(t, b).start()
            if t + NBUF < n_chunks:
                gather_copy(t + NBUF, b).start()

        for t in range(n_chunks - NBUF, n_chunks):
            store_copy(t, t % NBUF).wait()

    return emb_kernel


def kernel(x, lut):
    NB, T = x.shape
    B = NB * T
    lutp = jnp.pad(lut, ((0, 0), (0, W - D)))
    out = _make_emb_kernel(B)(x.reshape(B).astype(jnp.int32), lutp)
    return out[:, :D].reshape(NB, T, D)
